# baseline (speedup 1.0000x reference)
import jax
import jax.numpy as jnp
from jax import lax
from jax.experimental import pallas as pl
from jax.experimental.pallas import tpu as pltpu
from jax.experimental.pallas import tpu_sc as plsc

_VOCAB = 1000
_RT = 200
_CT = 16384
_NW = 32
_COLS_W = _CT // _NW
_CHUNKS = ((0, 8), (8, 24), (32, 24), (56, 24), (80, 24), (104, 24),
           (128, 24), (152, 24), (176, 16), (192, 8))
_NCHUNK = len(_CHUNKS)
_RBUF = 24
_NSLICE = _COLS_W // 16


_NBUF = 2


def _gather_kernel(index_hbm, table_hbm, out_hbm,
                   table_v, idx_v, out_v, idx_sp,
                   isemA0, isemA1, isemB0, isemB1, osem0, osem1):
    cid = lax.axis_index("c")
    sid = lax.axis_index("s")
    wid = sid * 2 + cid
    col0 = wid * _COLS_W

    def hbm2sp(c, b):
        r0, nr = _CHUNKS[c]
        return pltpu.async_copy(
            index_hbm.at[pl.ds(r0, nr), pl.ds(col0, _COLS_W)],
            idx_sp.at[sid, b, pl.ds(0, nr)], (isemA0, isemA1)[b])

    def sp2tile(c, b):
        nr = _CHUNKS[c][1]
        return pltpu.async_copy(
            idx_sp.at[sid, b, pl.ds(0, nr)],
            idx_v.at[b, pl.ds(0, nr)], (isemB0, isemB1)[b])

    def start_out(c, b):
        r0, nr = _CHUNKS[c]
        return pltpu.async_copy(
            out_v.at[b, pl.ds(0, nr)],
            out_hbm.at[pl.ds(r0, nr), pl.ds(col0, _COLS_W)],
            (osem0, osem1)[b])

    def compute(c, b):
        nr = _CHUNKS[c][1]

        def step(r, carry):
            idxs = [idx_v[b, r, pl.ds(o * 16, 16)] for o in range(_NSLICE)]
            vals = [plsc.load_gather(table_v, [ix]) for ix in idxs]
            for o, v in enumerate(vals):
                out_v[b, r, pl.ds(o * 16, 16)] = v
            return carry

        lax.fori_loop(0, nr, step, 0)

    s1 = {}
    s2 = {}
    out_handles = {}
    s1[0] = hbm2sp(0, 0)
    s1[1] = hbm2sp(1, 1)
    pltpu.sync_copy(table_hbm, table_v)
    s1[0].wait()
    s2[0] = sp2tile(0, 0)
    for c in range(_NCHUNK):
        b = c % _NBUF
        if c + 1 < _NCHUNK:
            s1[c + 1].wait()
            s2[c + 1] = sp2tile(c + 1, (c + 1) % _NBUF)
        s2[c].wait()
        if c >= _NBUF:
            out_handles[c - _NBUF].wait()
        compute(c, b)
        out_handles[c] = start_out(c, b)
        if c + _NBUF < _NCHUNK:
            s1[c + _NBUF] = hbm2sp(c + _NBUF, b)
    for c in range(_NCHUNK - _NBUF, _NCHUNK):
        out_handles[c].wait()


@jax.jit
def kernel(index, table):
    mesh = plsc.VectorSubcoreMesh(core_axis_name="c", subcore_axis_name="s")
    run = pl.kernel(
        _gather_kernel,
        out_type=jax.ShapeDtypeStruct((_RT, _CT), jnp.float32),
        mesh=mesh,
        scratch_types=[
            pltpu.VMEM((_VOCAB,), jnp.float32),
            pltpu.VMEM((_NBUF, _RBUF, _COLS_W), jnp.int32),
            pltpu.VMEM((_NBUF, _RBUF, _COLS_W), jnp.float32),
            pltpu.VMEM_SHARED((16, _NBUF, _RBUF, _COLS_W), jnp.int32),
            pltpu.SemaphoreType.DMA,
            pltpu.SemaphoreType.DMA,
            pltpu.SemaphoreType.DMA,
            pltpu.SemaphoreType.DMA,
            pltpu.SemaphoreType.DMA,
            pltpu.SemaphoreType.DMA,
        ],
        compiler_params=pltpu.CompilerParams(
            needs_layout_passes=False,
            use_tc_tiling_on_sc=True,
        ),
    )
    out_t = run(index.T, table)
    return out_t.T


# speedup vs baseline: 1.2400x; 1.2400x over previous
import jax
import jax.numpy as jnp
from jax import lax
from jax.experimental import pallas as pl
from jax.experimental.pallas import tpu as pltpu
from jax.experimental.pallas import tpu_sc as plsc

_VOCAB = 1000
_RT = 200
_CT = 16384
_NW = 32
_COLS_W = _CT // _NW
_CHUNKS = ((0, 8), (8, 48), (56, 48), (104, 48), (152, 40), (192, 8))
_NCHUNK = len(_CHUNKS)
_RBUF = 48
_NSLICE = _COLS_W // 16


_NBUF = 2


def _gather_kernel(index_hbm, table_hbm, out_hbm,
                   table_v, idx_v, out_v, isem0, isem1, osem0, osem1):
    wid = lax.axis_index("s") * 2 + lax.axis_index("c")
    col0 = wid * _COLS_W

    def start_in(c, b):
        r0, nr = _CHUNKS[c]
        return pltpu.async_copy(
            index_hbm.at[pl.ds(r0, nr), pl.ds(col0, _COLS_W)],
            idx_v.at[b, pl.ds(0, nr)], (isem0, isem1)[b])

    def start_out(c, b):
        r0, nr = _CHUNKS[c]
        return pltpu.async_copy(
            out_v.at[b, pl.ds(0, nr)],
            out_hbm.at[pl.ds(r0, nr), pl.ds(col0, _COLS_W)],
            (osem0, osem1)[b])

    def compute(c, b):
        nr = _CHUNKS[c][1]

        def step(r, carry):
            idxs = [idx_v[b, r, pl.ds(o * 16, 16)] for o in range(_NSLICE)]
            vals = [plsc.load_gather(table_v, [ix]) for ix in idxs]
            for o, v in enumerate(vals):
                out_v[b, r, pl.ds(o * 16, 16)] = v
            return carry

        lax.fori_loop(0, nr, step, 0)

    in_handles = {}
    out_handles = {}
    pltpu.sync_copy(table_hbm, table_v)
    for c in range(_NBUF):
        in_handles[c] = start_in(c, c)
    for c in range(_NCHUNK):
        b = c % _NBUF
        in_handles[c].wait()
        if c >= _NBUF:
            out_handles[c - _NBUF].wait()
        compute(c, b)
        out_handles[c] = start_out(c, b)
        if c + _NBUF < _NCHUNK:
            in_handles[c + _NBUF] = start_in(c + _NBUF, b)
    for c in range(_NCHUNK - _NBUF, _NCHUNK):
        out_handles[c].wait()


@jax.jit
def kernel(index, table):
    mesh = plsc.VectorSubcoreMesh(core_axis_name="c", subcore_axis_name="s")
    run = pl.kernel(
        _gather_kernel,
        out_type=jax.ShapeDtypeStruct((_RT, _CT), jnp.float32),
        mesh=mesh,
        scratch_types=[
            pltpu.VMEM((_VOCAB,), jnp.float32),
            pltpu.VMEM((_NBUF, _RBUF, _COLS_W), jnp.int32),
            pltpu.VMEM((_NBUF, _RBUF, _COLS_W), jnp.float32),
            pltpu.SemaphoreType.DMA,
            pltpu.SemaphoreType.DMA,
            pltpu.SemaphoreType.DMA,
            pltpu.SemaphoreType.DMA,
        ],
        compiler_params=pltpu.CompilerParams(
            needs_layout_passes=False,
            use_tc_tiling_on_sc=True,
        ),
    )
    out_t = run(index.T, table)
    return out_t.T


# speedup vs baseline: 1.2482x; 1.0066x over previous
import jax
import jax.numpy as jnp
from jax import lax
from jax.experimental import pallas as pl
from jax.experimental.pallas import tpu as pltpu
from jax.experimental.pallas import tpu_sc as plsc

_VOCAB = 1000
_RT = 200
_CT = 16384
_NW = 32
_COLS_W = _CT // _NW
_CHUNKS = ((0, 8), (8, 48), (56, 48), (104, 48), (152, 40), (192, 8))
_NCHUNK = len(_CHUNKS)
_RBUF = 48
_NSLICE = _COLS_W // 16


_NBUF = 2


def _gather_kernel(index_hbm, table_hbm, out_hbm,
                   table_v, idx_v, out_v, isem0, isem1, osem0, osem1):
    wid = lax.axis_index("s") * 2 + lax.axis_index("c")
    col0 = wid * _COLS_W

    def start_in(c, b):
        r0, nr = _CHUNKS[c]
        return pltpu.async_copy(
            index_hbm.at[pl.ds(r0, nr), pl.ds(col0, _COLS_W)],
            idx_v.at[b, pl.ds(0, nr)], (isem0, isem1)[b])

    def start_out(c, b):
        r0, nr = _CHUNKS[c]
        return pltpu.async_copy(
            out_v.at[b, pl.ds(0, nr)],
            out_hbm.at[pl.ds(r0, nr), pl.ds(col0, _COLS_W)],
            (osem0, osem1)[b])

    def compute(c, b):
        nr = _CHUNKS[c][1]

        def step(r, carry):
            idxs = [idx_v[b, r, pl.ds(o * 16, 16)] for o in range(_NSLICE)]
            vals = [plsc.load_gather(table_v, [ix]) for ix in idxs]
            for o, v in enumerate(vals):
                out_v[b, r, pl.ds(o * 16, 16)] = v
            return carry

        lax.fori_loop(0, nr, step, 0)

    in_handles = {}
    out_handles = {}
    for c in range(_NBUF):
        in_handles[c] = start_in(c, c)
    pltpu.sync_copy(table_hbm, table_v)
    for c in range(_NCHUNK):
        b = c % _NBUF
        in_handles[c].wait()
        if c >= _NBUF:
            out_handles[c - _NBUF].wait()
        compute(c, b)
        out_handles[c] = start_out(c, b)
        if c + _NBUF < _NCHUNK:
            in_handles[c + _NBUF] = start_in(c + _NBUF, b)
    for c in range(_NCHUNK - _NBUF, _NCHUNK):
        out_handles[c].wait()


@jax.jit
def kernel(index, table):
    mesh = plsc.VectorSubcoreMesh(core_axis_name="c", subcore_axis_name="s")
    run = pl.kernel(
        _gather_kernel,
        out_type=jax.ShapeDtypeStruct((_RT, _CT), jnp.float32),
        mesh=mesh,
        scratch_types=[
            pltpu.VMEM((_VOCAB,), jnp.float32),
            pltpu.VMEM((_NBUF, _RBUF, _COLS_W), jnp.int32),
            pltpu.VMEM((_NBUF, _RBUF, _COLS_W), jnp.float32),
            pltpu.SemaphoreType.DMA,
            pltpu.SemaphoreType.DMA,
            pltpu.SemaphoreType.DMA,
            pltpu.SemaphoreType.DMA,
        ],
        compiler_params=pltpu.CompilerParams(
            needs_layout_passes=False,
            use_tc_tiling_on_sc=True,
        ),
    )
    out_t = run(index.T, table)
    return out_t.T
